# baseline (speedup 1.0000x reference)
import jax
import jax.numpy as jnp
from jax.experimental import pallas as pl
from jax.experimental.pallas import tpu as pltpu

BS, T, C_IN = 4, 8, 256
K = 17
NODE_DIM = 1024
HID = 256
PRED_PAD = 128

_PREC = jax.lax.Precision.HIGHEST


def _frame_body(x_ref, A_ref, Wc_ref, bc_ref, Wg_ref, bg_ref, out_ref):
    xb = x_ref[0]
    nodes = jnp.dot(Wc_ref[:], xb, precision=_PREC,
                    preferred_element_type=jnp.float32) + bc_ref[:]
    g = jnp.dot(A_ref[:], nodes, precision=_PREC,
                preferred_element_type=jnp.float32)
    gw = jnp.dot(g, Wg_ref[:], precision=_PREC,
                 preferred_element_type=jnp.float32) + bg_ref[:]
    out_ref[0, 0] = jnp.maximum(gw, 0.0)


def _gru_body(s_ref, Wz_ref, Uz_ref, bz_ref, Wr_ref, Ur_ref, br_ref,
              Wn_ref, Un_ref, bn_ref, Wp_ref, bp_ref, feat_ref, pred_ref):
    rows = BS * K

    def step(t, h):
        xt = s_ref[pl.ds(t, 1)][0]
        z = jax.nn.sigmoid(
            jnp.dot(xt, Wz_ref[:], precision=_PREC,
                    preferred_element_type=jnp.float32)
            + jnp.dot(h, Uz_ref[:], precision=_PREC,
                      preferred_element_type=jnp.float32)
            + bz_ref[:])
        r = jax.nn.sigmoid(
            jnp.dot(xt, Wr_ref[:], precision=_PREC,
                    preferred_element_type=jnp.float32)
            + jnp.dot(h, Ur_ref[:], precision=_PREC,
                      preferred_element_type=jnp.float32)
            + br_ref[:])
        n = jnp.tanh(
            jnp.dot(xt, Wn_ref[:], precision=_PREC,
                    preferred_element_type=jnp.float32)
            + r * jnp.dot(h, Un_ref[:], precision=_PREC,
                          preferred_element_type=jnp.float32)
            + bn_ref[:])
        hn = (1.0 - z) * h + z * n
        feat_ref[pl.ds(t, 1)] = hn[None]
        pred_ref[pl.ds(t, 1)] = (jnp.dot(hn, Wp_ref[:], precision=_PREC,
                                         preferred_element_type=jnp.float32)
                                 + bp_ref[:])[None]
        return hn

    jax.lax.fori_loop(0, T, step, jnp.zeros((rows, HID), jnp.float32))


def kernel(x, A, Wconv, bconv, Wg, bg, Wz, Uz, bz, Wr, Ur, br,
           Wn, Un, bn, Wp, bp):
    b, t, c, h, w = x.shape
    xf = x.reshape(b * t, c, h * w)

    seq = pl.pallas_call(
        _frame_body,
        grid=(b * t,),
        in_specs=[
            pl.BlockSpec((1, c, h * w), lambda i: (i, 0, 0)),
            pl.BlockSpec((K, K), lambda i: (0, 0)),
            pl.BlockSpec((K, c), lambda i: (0, 0)),
            pl.BlockSpec((K, 1), lambda i: (0, 0)),
            pl.BlockSpec((h * w, HID), lambda i: (0, 0)),
            pl.BlockSpec((1, HID), lambda i: (0, 0)),
        ],
        out_specs=pl.BlockSpec((1, 1, K, HID), lambda i: (i % t, i // t, 0, 0)),
        out_shape=jax.ShapeDtypeStruct((t, b, K, HID), jnp.float32),
        compiler_params=pltpu.CompilerParams(
            dimension_semantics=("arbitrary",)),
    )(xf, A, Wconv, bconv.reshape(K, 1), Wg, bg.reshape(1, HID))
    seq = seq.reshape(t, b * K, HID)

    Wp_pad = jnp.zeros((HID, PRED_PAD), jnp.float32).at[:, :2].set(Wp)
    bp_pad = jnp.zeros((1, PRED_PAD), jnp.float32).at[:, :2].set(bp)

    feat_t, pred_t = pl.pallas_call(
        _gru_body,
        out_shape=[
            jax.ShapeDtypeStruct((t, b * K, HID), jnp.float32),
            jax.ShapeDtypeStruct((t, b * K, PRED_PAD), jnp.float32),
        ],
    )(seq, Wz, Uz, bz.reshape(1, HID), Wr, Ur, br.reshape(1, HID),
      Wn, Un, bn.reshape(1, HID), Wp_pad, bp_pad)

    feat = feat_t.reshape(t, b, K, HID).transpose(1, 0, 2, 3)
    pred = pred_t[..., :2].reshape(t, b, K, 2).transpose(1, 0, 2, 3)
    return pred, feat


# speedup vs baseline: 1.3261x; 1.3261x over previous
import jax
import jax.numpy as jnp
from jax.experimental import pallas as pl
from jax.experimental.pallas import tpu as pltpu

BS, T, C_IN = 4, 8, 256
K = 17
NODE_DIM = 1024
HID = 256
PRED_PAD = 128

_PREC = None


def _frame_body(x_ref, A_ref, Wc_ref, bc_ref, Wg_ref, bg_ref,
                Wzrn_ref, bzrn_ref, out_ref):
    AW = jnp.dot(A_ref[:], Wc_ref[:], precision=_PREC,
                 preferred_element_type=jnp.float32)
    ab = jnp.dot(A_ref[:], bc_ref[:], precision=_PREC,
                 preferred_element_type=jnp.float32)
    g = jnp.dot(AW, x_ref[0], precision=_PREC,
                preferred_element_type=jnp.float32) + ab
    gw = jnp.maximum(
        jnp.dot(g, Wg_ref[:], precision=_PREC,
                preferred_element_type=jnp.float32) + bg_ref[:], 0.0)
    out_ref[0, 0] = jnp.dot(gw, Wzrn_ref[:], precision=_PREC,
                            preferred_element_type=jnp.float32) + bzrn_ref[:]


def _gru_body(s_ref, Uzrn_ref, Wp_ref, bp_ref, feat_ref, pred_ref):
    rows = BS * K

    def step(t, h):
        xp = s_ref[pl.ds(t, 1)][0]
        hu = jnp.dot(h, Uzrn_ref[:], precision=_PREC,
                     preferred_element_type=jnp.float32)
        z = jax.nn.sigmoid(xp[:, :HID] + hu[:, :HID])
        r = jax.nn.sigmoid(xp[:, HID:2 * HID] + hu[:, HID:2 * HID])
        n = jnp.tanh(xp[:, 2 * HID:] + r * hu[:, 2 * HID:])
        hn = h + z * (n - h)
        feat_ref[pl.ds(t, 1)] = hn[None]
        pred_ref[pl.ds(t, 1)] = (jnp.dot(hn, Wp_ref[:], precision=_PREC,
                                         preferred_element_type=jnp.float32)
                                 + bp_ref[:])[None]
        return hn

    jax.lax.fori_loop(0, T, step, jnp.zeros((rows, HID), jnp.float32))


def kernel(x, A, Wconv, bconv, Wg, bg, Wz, Uz, bz, Wr, Ur, br,
           Wn, Un, bn, Wp, bp):
    b, t, c, h, w = x.shape
    xf = x.reshape(b * t, c, h * w)

    Wzrn = jnp.concatenate([Wz, Wr, Wn], axis=1)
    bzrn = jnp.concatenate([bz, br, bn]).reshape(1, 3 * HID)
    Uzrn = jnp.concatenate([Uz, Ur, Un], axis=1)

    sp = pl.pallas_call(
        _frame_body,
        grid=(b * t,),
        in_specs=[
            pl.BlockSpec((1, c, h * w), lambda i: (i, 0, 0)),
            pl.BlockSpec((K, K), lambda i: (0, 0)),
            pl.BlockSpec((K, c), lambda i: (0, 0)),
            pl.BlockSpec((K, 1), lambda i: (0, 0)),
            pl.BlockSpec((h * w, HID), lambda i: (0, 0)),
            pl.BlockSpec((1, HID), lambda i: (0, 0)),
            pl.BlockSpec((HID, 3 * HID), lambda i: (0, 0)),
            pl.BlockSpec((1, 3 * HID), lambda i: (0, 0)),
        ],
        out_specs=pl.BlockSpec((1, 1, K, 3 * HID),
                               lambda i: (i % t, i // t, 0, 0)),
        out_shape=jax.ShapeDtypeStruct((t, b, K, 3 * HID), jnp.float32),
        compiler_params=pltpu.CompilerParams(
            dimension_semantics=("arbitrary",)),
    )(xf, A, Wconv, bconv.reshape(K, 1), Wg, bg.reshape(1, HID), Wzrn, bzrn)
    sp = sp.reshape(t, b * K, 3 * HID)

    Wp_pad = jnp.zeros((HID, PRED_PAD), jnp.float32).at[:, :2].set(Wp)
    bp_pad = jnp.zeros((1, PRED_PAD), jnp.float32).at[:, :2].set(bp)

    feat_t, pred_t = pl.pallas_call(
        _gru_body,
        out_shape=[
            jax.ShapeDtypeStruct((t, b * K, HID), jnp.float32),
            jax.ShapeDtypeStruct((t, b * K, PRED_PAD), jnp.float32),
        ],
    )(sp, Uzrn, Wp_pad, bp_pad)

    feat = feat_t.reshape(t, b, K, HID).transpose(1, 0, 2, 3)
    pred = pred_t[..., :2].reshape(t, b, K, 2).transpose(1, 0, 2, 3)
    return pred, feat


# speedup vs baseline: 1.6184x; 1.2204x over previous
import jax
import jax.numpy as jnp
from jax.experimental import pallas as pl
from jax.experimental.pallas import tpu as pltpu

BS, T, C_IN = 4, 8, 256
K = 17
NODE_DIM = 1024
HID = 256
PRED_PAD = 128

_PREC = None


def _fused_body(x0_ref, x1_ref, x2_ref, x3_ref, A_ref, Wc_ref, bc_ref,
                Wg_ref, bg_ref, Wzrn_ref, bzrn_ref, Uzrn_ref, Wp_ref, bp_ref,
                feat_ref, pred_ref, h_scr):
    x_refs = (x0_ref, x1_ref, x2_ref, x3_ref)
    j = pl.program_id(0)

    @pl.when(j == 0)
    def _init():
        h_scr[...] = jnp.zeros_like(h_scr)

    AW = jnp.dot(A_ref[:], Wc_ref[:], precision=_PREC,
                 preferred_element_type=jnp.float32)
    ab = jnp.dot(A_ref[:], bc_ref[:], precision=_PREC,
                 preferred_element_type=jnp.float32)

    for b in range(BS):
        g = jnp.dot(AW, x_refs[b][0], precision=_PREC,
                    preferred_element_type=jnp.float32) + ab
        gw = jnp.maximum(
            jnp.dot(g, Wg_ref[:], precision=_PREC,
                    preferred_element_type=jnp.float32) + bg_ref[:], 0.0)
        xp = jnp.dot(gw, Wzrn_ref[:], precision=_PREC,
                     preferred_element_type=jnp.float32) + bzrn_ref[:]
        h = h_scr[b]
        hu = jnp.dot(h, Uzrn_ref[:], precision=_PREC,
                     preferred_element_type=jnp.float32)
        z = jax.nn.sigmoid(xp[:, :HID] + hu[:, :HID])
        r = jax.nn.sigmoid(xp[:, HID:2 * HID] + hu[:, HID:2 * HID])
        n = jnp.tanh(xp[:, 2 * HID:] + r * hu[:, 2 * HID:])
        hn = h + z * (n - h)
        h_scr[b] = hn
        feat_ref[0, b * K:(b + 1) * K] = hn
        pred_ref[0, b * K:(b + 1) * K] = jnp.dot(
            hn, Wp_ref[:], precision=_PREC,
            preferred_element_type=jnp.float32) + bp_ref[:]


def kernel(x, A, Wconv, bconv, Wg, bg, Wz, Uz, bz, Wr, Ur, br,
           Wn, Un, bn, Wp, bp):
    b, t, c, h, w = x.shape
    xf = x.reshape(b * t, c, h * w)

    Wzrn = jnp.concatenate([Wz, Wr, Wn], axis=1)
    bzrn = jnp.concatenate([bz, br, bn]).reshape(1, 3 * HID)
    Uzrn = jnp.concatenate([Uz, Ur, Un], axis=1)
    Wp_pad = jnp.zeros((HID, PRED_PAD), jnp.float32).at[:, :2].set(Wp)
    bp_pad = jnp.zeros((1, PRED_PAD), jnp.float32).at[:, :2].set(bp)

    def _xspec(bb):
        return pl.BlockSpec((1, c, h * w), lambda j, bb=bb: (bb * t + j, 0, 0))

    feat_t, pred_t = pl.pallas_call(
        _fused_body,
        grid=(t,),
        in_specs=[
            _xspec(0), _xspec(1), _xspec(2), _xspec(3),
            pl.BlockSpec((K, K), lambda j: (0, 0)),
            pl.BlockSpec((K, c), lambda j: (0, 0)),
            pl.BlockSpec((K, 1), lambda j: (0, 0)),
            pl.BlockSpec((h * w, HID), lambda j: (0, 0)),
            pl.BlockSpec((1, HID), lambda j: (0, 0)),
            pl.BlockSpec((HID, 3 * HID), lambda j: (0, 0)),
            pl.BlockSpec((1, 3 * HID), lambda j: (0, 0)),
            pl.BlockSpec((HID, 3 * HID), lambda j: (0, 0)),
            pl.BlockSpec((HID, PRED_PAD), lambda j: (0, 0)),
            pl.BlockSpec((1, PRED_PAD), lambda j: (0, 0)),
        ],
        out_specs=[
            pl.BlockSpec((1, BS * K, HID), lambda j: (j, 0, 0)),
            pl.BlockSpec((1, BS * K, PRED_PAD), lambda j: (j, 0, 0)),
        ],
        out_shape=[
            jax.ShapeDtypeStruct((t, BS * K, HID), jnp.float32),
            jax.ShapeDtypeStruct((t, BS * K, PRED_PAD), jnp.float32),
        ],
        scratch_shapes=[pltpu.VMEM((BS, K, HID), jnp.float32)],
        compiler_params=pltpu.CompilerParams(
            dimension_semantics=("arbitrary",)),
    )(xf, xf, xf, xf, A, Wconv, bconv.reshape(K, 1), Wg, bg.reshape(1, HID),
      Wzrn, bzrn, Uzrn, Wp_pad, bp_pad)

    feat = feat_t.reshape(t, b, K, HID).transpose(1, 0, 2, 3)
    pred = pred_t[..., :2].reshape(t, b, K, 2).transpose(1, 0, 2, 3)
    return pred, feat
